# baseline (speedup 1.0000x reference)
import jax
import jax.numpy as jnp
from jax import lax
from jax.experimental import pallas as pl
from jax.experimental.pallas import tpu as pltpu
from jax.experimental.pallas import tpu_sc as plsc

N_TOK = 16384
DIM = 4096
LANES = 16
NC = 2
NS = 16
NW = NC * NS
ROWS_PER_W = N_TOK // NW
CHUNK = 8
N_CHUNK = ROWS_PER_W // CHUNK
NBUF = 3


def _sc_body(x_hbm, a_hbm, o_hbm, a_v, bufs, sis, sos):
    wid = lax.axis_index("s") * NC + lax.axis_index("c")
    base = wid * ROWS_PER_W
    pltpu.sync_copy(a_hbm, a_v)

    def in_slice(c):
        return x_hbm.at[pl.ds(base + c * CHUNK, CHUNK)]

    def out_slice(c):
        return o_hbm.at[pl.ds(base + c * CHUNK, CHUNK)]

    pltpu.async_copy(in_slice(0), bufs[0], sis[0])

    def step(c, b):
        b1 = (b + 1) % NBUF
        buf, si, so = bufs[b], sis[b], sos[b]

        @pl.when(c + 1 < N_CHUNK)
        def _():
            @pl.when(c - 2 >= 0)
            def _():
                pltpu.make_async_copy(bufs[b1], out_slice(c - 2), sos[b1]).wait()
            pltpu.async_copy(in_slice(c + 1), bufs[b1], sis[b1])

        pltpu.make_async_copy(in_slice(c), buf, si).wait()

        @plsc.parallel_loop(0, DIM // LANES, unroll=8)
        def col_body(k):
            a_reg = a_v[pl.ds(k * LANES, LANES)]
            for r in range(CHUNK):
                buf[r, pl.ds(k * LANES, LANES)] = (
                    buf[r, pl.ds(k * LANES, LANES)] * a_reg
                )

        pltpu.async_copy(buf, out_slice(c), so)

    def tri_body(c3, _):
        for b in range(NBUF):
            step(c3 * NBUF + b, b)
        return 0

    lax.fori_loop(0, (N_CHUNK - 1) // NBUF, tri_body, 0)
    step(N_CHUNK - 1, (N_CHUNK - 1) % NBUF)

    for c in (N_CHUNK - 3, N_CHUNK - 2, N_CHUNK - 1):
        b = c % NBUF
        pltpu.make_async_copy(bufs[b], out_slice(c), sos[b]).wait()


def kernel(x, a):
    mesh = plsc.VectorSubcoreMesh(core_axis_name="c", subcore_axis_name="s")
    f = pl.kernel(
        _sc_body,
        out_type=jax.ShapeDtypeStruct((N_TOK, DIM), jnp.float32),
        mesh=mesh,
        scratch_types=[
            pltpu.VMEM((DIM,), jnp.float32),
            [pltpu.VMEM((CHUNK, DIM), jnp.float32) for _ in range(NBUF)],
            [pltpu.SemaphoreType.DMA for _ in range(NBUF)],
            [pltpu.SemaphoreType.DMA for _ in range(NBUF)],
        ],
    )
    return f(x, a)


# speedup vs baseline: 1.2676x; 1.2676x over previous
import jax
import jax.numpy as jnp
from jax.experimental import pallas as pl
from jax.experimental.pallas import tpu as pltpu

N_TOK = 16384
DIM = 4096
BLOCK_ROWS = 1016


def _scale_body(a_ref, x_ref, o_ref):
    o_ref[...] = x_ref[...] * a_ref[...]


def kernel(x, a):
    a2 = a.reshape(1, DIM)
    grid = (pl.cdiv(N_TOK, BLOCK_ROWS),)
    return pl.pallas_call(
        _scale_body,
        grid=grid,
        in_specs=[
            pl.BlockSpec((1, DIM), lambda i: (0, 0)),
            pl.BlockSpec((BLOCK_ROWS, DIM), lambda i: (i, 0)),
        ],
        out_specs=pl.BlockSpec((BLOCK_ROWS, DIM), lambda i: (i, 0)),
        out_shape=jax.ShapeDtypeStruct((N_TOK, DIM), jnp.float32),
        compiler_params=pltpu.CompilerParams(
            dimension_semantics=("parallel",),
            vmem_limit_bytes=100 * 1024 * 1024,
        ),
    )(a2, x)
